# baseline (speedup 1.0000x reference)
import functools

import jax
import jax.numpy as jnp
from jax import lax
from jax.experimental import pallas as pl
from jax.experimental.pallas import tpu as pltpu
from jax.experimental.pallas import tpu_sc as plsc

VOCAB = 1000
VPAD = 1024
NSL = VPAD // 128
N_TOK = 1024 * 200

NC, NS, L = 2, 16, 16
NW = NC * NS
B_PER_W = N_TOK // NW
CHUNK = 32
N_CHUNKS = B_PER_W // CHUNK

RT_BLK = 512


def _lse_body(t_ref, o_ref):
    x = t_ref[...]
    m = jnp.max(x, axis=1, keepdims=True)
    o_ref[...] = jnp.log(jnp.sum(jnp.exp(x - m), axis=1, keepdims=True)) + m


def _row_lse(table):
    return pl.pallas_call(
        _lse_body,
        out_shape=jax.ShapeDtypeStruct((VOCAB, 1), jnp.float32),
    )(table).reshape(VOCAB)


def _retile_body(x_ref, o_ref):
    for c in range(NSL - 1):
        o_ref[:, pl.ds(c * 128, 128)] = x_ref[:, c, :]
    o_ref[:, pl.ds((NSL - 1) * 128, VOCAB - (NSL - 1) * 128)] = (
        x_ref[:, NSL - 1, : VOCAB - (NSL - 1) * 128])


def _retile(x3):
    return pl.pallas_call(
        _retile_body,
        grid=(N_TOK // RT_BLK,),
        in_specs=[pl.BlockSpec((RT_BLK, NSL, 128), lambda i: (i, 0, 0))],
        out_specs=pl.BlockSpec((RT_BLK, VOCAB), lambda i: (i, 0)),
        out_shape=jax.ShapeDtypeStruct((N_TOK, VOCAB), jnp.float32),
    )(x3)


@functools.partial(
    pl.kernel,
    out_type=[
        jax.ShapeDtypeStruct((N_TOK, NSL, 128), jnp.float32),
        jax.ShapeDtypeStruct((NW * L,), jnp.float32),
    ],
    mesh=plsc.VectorSubcoreMesh(core_axis_name="c", subcore_axis_name="s"),
    compiler_params=pltpu.CompilerParams(
        use_tc_tiling_on_sc=True,
        needs_layout_passes=False,
    ),
    scratch_types=[
        pltpu.VMEM((VOCAB,), jnp.float32),
        pltpu.VMEM((CHUNK,), jnp.int32),
        pltpu.VMEM((CHUNK,), jnp.int32),
        pltpu.VMEM((CHUNK,), jnp.int32),
        pltpu.VMEM((CHUNK,), jnp.int32),
        pltpu.VMEM((CHUNK,), jnp.float32),
        pltpu.VMEM((CHUNK,), jnp.float32),
        pltpu.VMEM((CHUNK, NSL, 128), jnp.float32),
        pltpu.VMEM((CHUNK, NSL, 128), jnp.float32),
        pltpu.VMEM((L,), jnp.float32),
        pltpu.SemaphoreType.DMA,
        pltpu.SemaphoreType.DMA,
        pltpu.SemaphoreType.DMA,
        pltpu.SemaphoreType.DMA,
        pltpu.SemaphoreType.DMA,
        pltpu.SemaphoreType.DMA,
    ],
)
def _sc_gather_loss(idx_hbm, tgt_hbm, table_hbm, tflat_hbm, lse_hbm,
                    out_hbm, part_hbm,
                    lse_v, idx0, idx1, fl0, fl1, tv0, tv1, rows0, rows1,
                    acc_v, sg0, sg1, sw0, sw1, st0, st1):
    wid = lax.axis_index("s") * NC + lax.axis_index("c")
    base = wid * B_PER_W
    bufs = ((idx0, fl0, tv0, rows0, sg0, sw0, st0),
            (idx1, fl1, tv1, rows1, sg1, sw1, st1))

    pltpu.sync_copy(lse_hbm, lse_v)
    acc_v[...] = jnp.zeros((L,), jnp.float32)

    def load_itgt(c, idx_b, fl_b, tv_b, st_b):
        off = base + c * CHUNK
        pltpu.sync_copy(idx_hbm.at[pl.ds(off, CHUNK)], idx_b)
        pltpu.sync_copy(tgt_hbm.at[pl.ds(off, CHUNK)], fl_b)
        for j in range(CHUNK // L):
            s = pl.ds(j * L, L)
            fl_b[s] = fl_b[s] + idx_b[s] * VPAD
        pltpu.async_copy(tflat_hbm.at[fl_b], tv_b, st_b)

    def loss(idx_b, fl_b, tv_b, st_b):
        pltpu.make_async_copy(tflat_hbm.at[fl_b], tv_b, st_b).wait()
        for j in range(CHUNK // L):
            s = pl.ds(j * L, L)
            lse_g = plsc.load_gather(lse_v, [idx_b[s]])
            acc_v[...] = acc_v[...] + (lse_g - tv_b[s])

    def visit(c, b, first):
        idx_b, fl_b, tv_b, rows_b, sg_b, sw_b, st_b = bufs[b]
        idx_o, fl_o, tv_o, rows_o, sg_o, sw_o, st_o = bufs[1 - b]
        if not first:
            pltpu.make_async_copy(
                rows_o, out_hbm.at[pl.ds(base, CHUNK)], sw_o).wait()

        @pl.when(c + 1 < N_CHUNKS)
        def _():
            pltpu.async_copy(table_hbm.at[idx_o], rows_o, sg_o)

        pltpu.make_async_copy(table_hbm.at[idx_b], rows_b, sg_b).wait()
        loss(idx_b, fl_b, tv_b, st_b)
        pltpu.async_copy(
            rows_b, out_hbm.at[pl.ds(base + c * CHUNK, CHUNK)], sw_b)

        @pl.when(c + 2 < N_CHUNKS)
        def _():
            load_itgt(c + 2, idx_b, fl_b, tv_b, st_b)

    load_itgt(0, idx0, fl0, tv0, st0)
    load_itgt(1, idx1, fl1, tv1, st1)
    pltpu.async_copy(table_hbm.at[idx0], rows0, sg0)
    visit(jnp.int32(0), 0, True)
    visit(jnp.int32(1), 1, False)

    def pair(p, carry):
        visit(2 * p, 0, False)
        visit(2 * p + 1, 1, False)
        return carry

    lax.fori_loop(1, N_CHUNKS // 2, pair, 0)
    pltpu.make_async_copy(
        rows1, out_hbm.at[pl.ds(base, CHUNK)], sw1).wait()
    pltpu.sync_copy(acc_v, part_hbm.at[pl.ds(wid * L, L)])


def kernel(idx, targets, table):
    idx_f = idx.reshape(N_TOK).astype(jnp.int32)
    tgt_f = targets.reshape(N_TOK).astype(jnp.int32)
    tbl_pad = jnp.pad(table, ((0, 0), (0, VPAD - VOCAB)))
    tbl3 = tbl_pad.reshape(VOCAB, NSL, 128)
    tbl_flat = tbl_pad.reshape(VOCAB * VPAD)
    lse = _row_lse(table)
    out3, parts = _sc_gather_loss(idx_f, tgt_f, tbl3, tbl_flat, lse)
    logits2 = _retile(out3)
    loss = jnp.sum(parts) / jnp.float32(N_TOK)
    return (logits2, loss)


# speedup vs baseline: 1.7085x; 1.7085x over previous
import functools

import jax
import jax.numpy as jnp
from jax import lax
from jax.experimental import pallas as pl
from jax.experimental.pallas import tpu as pltpu
from jax.experimental.pallas import tpu_sc as plsc

VOCAB = 1000
VPAD = 1024
N_TOK = 1024 * 200

NC, NS, L = 2, 16, 16
NW = NC * NS
B_PER_W = N_TOK // NW
CHUNK = 32
N_CHUNKS = B_PER_W // CHUNK
TV_P = 128


def _lse_body(t_ref, o_ref):
    x = t_ref[...]
    m = jnp.max(x, axis=1, keepdims=True)
    o_ref[...] = jnp.log(jnp.sum(jnp.exp(x - m), axis=1, keepdims=True)) + m


def _row_lse(table):
    return pl.pallas_call(
        _lse_body,
        out_shape=jax.ShapeDtypeStruct((VOCAB, 1), jnp.float32),
    )(table).reshape(VOCAB)


@functools.partial(
    pl.kernel,
    out_type=[
        jax.ShapeDtypeStruct((N_TOK, VPAD), jnp.float32),
        jax.ShapeDtypeStruct((NW * L,), jnp.float32),
    ],
    mesh=plsc.VectorSubcoreMesh(core_axis_name="c", subcore_axis_name="s"),
    compiler_params=pltpu.CompilerParams(
        use_tc_tiling_on_sc=True,
        needs_layout_passes=False,
    ),
    scratch_types=[
        pltpu.VMEM((VOCAB,), jnp.float32),
        pltpu.VMEM((B_PER_W,), jnp.int32),
        pltpu.VMEM((B_PER_W,), jnp.int32),
        pltpu.VMEM((B_PER_W,), jnp.float32),
        pltpu.VMEM((CHUNK, VPAD), jnp.float32),
        pltpu.VMEM((CHUNK, VPAD), jnp.float32),
        pltpu.VMEM((L,), jnp.float32),
        pltpu.SemaphoreType.DMA,
        pltpu.SemaphoreType.DMA,
        pltpu.SemaphoreType.DMA,
        pltpu.SemaphoreType.DMA,
        pltpu.SemaphoreType.DMA,
    ],
)
def _sc_gather_loss(idx_hbm, tgt_hbm, table_hbm, tflat_hbm, lse_hbm,
                    out_hbm, part_hbm,
                    lse_v, idxa, fla, tva, rows0, rows1,
                    acc_v, sg0, sg1, sw0, sw1, stv):
    wid = lax.axis_index("s") * NC + lax.axis_index("c")
    base = wid * B_PER_W
    bufs = ((rows0, sg0, sw0), (rows1, sg1, sw1))

    pltpu.sync_copy(idx_hbm.at[pl.ds(base, B_PER_W)], idxa)
    pltpu.sync_copy(tgt_hbm.at[pl.ds(base, B_PER_W)], fla)
    pltpu.sync_copy(lse_hbm, lse_v)

    def _flat(i, c):
        s = pl.ds(i * L, L)
        fla[s] = fla[s] + idxa[s] * VPAD
        return c

    lax.fori_loop(0, B_PER_W // L, _flat, 0)

    pltpu.async_copy(table_hbm.at[idxa.at[pl.ds(0, CHUNK)]], rows0, sg0)
    pltpu.async_copy(table_hbm.at[idxa.at[pl.ds(CHUNK, CHUNK)]], rows1, sg1)

    for i in range(B_PER_W // TV_P):
        p = pl.ds(i * TV_P, TV_P)
        pltpu.async_copy(tflat_hbm.at[fla.at[p]], tva.at[p], stv)

    def visit(c, b, first):
        rows_b, sg_b, sw_b = bufs[b]
        rows_o, sg_o, sw_o = bufs[1 - b]
        if not first:
            pltpu.make_async_copy(
                rows_o, out_hbm.at[pl.ds(base, CHUNK)], sw_o).wait()

        @pl.when(c + 1 < N_CHUNKS)
        def _():
            pltpu.async_copy(
                table_hbm.at[idxa.at[pl.ds((c + 1) * CHUNK, CHUNK)]],
                rows_o, sg_o)

        pltpu.make_async_copy(
            table_hbm.at[idxa.at[pl.ds(c * CHUNK, CHUNK)]], rows_b,
            sg_b).wait()
        pltpu.async_copy(
            rows_b, out_hbm.at[pl.ds(base + c * CHUNK, CHUNK)], sw_b)

    visit(jnp.int32(0), 0, True)
    visit(jnp.int32(1), 1, False)

    def pair(p, carry):
        visit(2 * p, 0, False)
        visit(2 * p + 1, 1, False)
        return carry

    lax.fori_loop(1, N_CHUNKS // 2, pair, 0)
    pltpu.make_async_copy(
        rows1, out_hbm.at[pl.ds(base, CHUNK)], sw1).wait()

    pltpu.make_async_copy(tflat_hbm.at[fla], tva, stv).wait()
    acc_v[...] = jnp.zeros((L,), jnp.float32)

    def _loss(i, c):
        s = pl.ds(i * L, L)
        lse_g = plsc.load_gather(lse_v, [idxa[s]])
        acc_v[...] = acc_v[...] + (lse_g - tva[s])
        return c

    lax.fori_loop(0, B_PER_W // L, _loss, 0)
    pltpu.sync_copy(acc_v, part_hbm.at[pl.ds(wid * L, L)])


def kernel(idx, targets, table):
    idx_f = idx.reshape(N_TOK).astype(jnp.int32)
    tgt_f = targets.reshape(N_TOK).astype(jnp.int32)
    tbl_pad = jnp.pad(table, ((0, 0), (0, VPAD - VOCAB)))
    tbl_flat = tbl_pad.reshape(VOCAB * VPAD)
    lse = _row_lse(table)
    out_pad, parts = _sc_gather_loss(idx_f, tgt_f, tbl_pad, tbl_flat, lse)
    logits2 = out_pad[:, :VOCAB]
    loss = jnp.sum(parts) / jnp.float32(N_TOK)
    return (logits2, loss)
